# baseline (speedup 1.0000x reference)
import functools
import numpy as np
import jax
import jax.numpy as jnp
from jax import lax
from jax.experimental import pallas as pl
from jax.experimental.pallas import tpu as pltpu, tpu_sc as plsc

NC, NS, L = 2, 16, 16
NW = NC * NS

PAD = 64

LEVELS = [
    (256, 48, 64, 8, 48, 16),
    (128, 96, 128, 4, 6, 128),
    (64, 192, 256, 2, 12, 64),
    (32, 384, 512, 1, 24, 32),
    (1, 384, 512, 1, 24, 1),
]

ORI_MAX = max(nr * w for (_, _, w, _, nr, _) in LEVELS)
MV_MAX = ORI_MAX
OSTR = ORI_MAX
SSTR = PAD + ORI_MAX


def _sc_body(o0, s0, o1, s1, o2, s2, o3, s3, od, sd,
             m0, m1, m2, mf, shsw, out,
             ori_v, shf_v, mv_v, sc_v, acc_v,
             sem_o0, sem_o1, sem_s0, sem_s1):
  wid = lax.axis_index("s") * NC + lax.axis_index("c")
  pair = wid // NS
  t = wid % NS
  lanes = lax.iota(jnp.int32, L)
  zeros = jnp.zeros((L,), jnp.float32)

  pltpu.sync_copy(shsw.at[pl.ds(pair * L, L)], sc_v)
  scv = sc_v[...]

  def get_scal(k):
    return scv[k]

  for b in range(2):
    for i in range(PAD // L):
      shf_v[pl.ds(b * SSTR + i * L, L)] = zeros
  sem_o = (sem_o0, sem_o1)
  sem_s = (sem_s0, sem_s1)

  def do_level(ori_h, shf_h, msk_h, lvl, num_acc, cnt_acc):
    c, h, w, s, nr, ct = LEVELS[lvl]
    nxb = w // L
    sh = get_scal(2 * lvl)
    sw = get_scal(2 * lvl + 1)
    if ct == c:
      c0 = 0
      r0 = t * nr
    else:
      c0 = t * ct
      r0 = 0
    start = jnp.maximum(r0 - sh, 0)
    d_off = r0 - sh - start

    pltpu.sync_copy(msk_h.at[pl.ds((pair * h + r0) * w, nr * w)],
                    ori_v.at[pl.ds(0, nr * w)])

    def mv_row(r, cacc):
      y = r0 + r
      rowf = jnp.where(y >= sh, jnp.float32(1.0), jnp.float32(0.0))
      cacc = list(cacc)
      for xb in range(nxb):
        xs = lanes + (xb * L)
        mval = ori_v[pl.ds(r * w + xb * L, L)]
        colv = jnp.where(xs >= sw, jnp.float32(1.0), jnp.float32(0.0))
        mv = mval * colv * rowf
        mv_v[pl.ds(r * w + xb * L, L)] = mv
        cacc[xb & 3] = cacc[xb & 3] + mv
      return tuple(cacc)

    mcnt = lax.fori_loop(0, nr, mv_row, (zeros,) * 4)
    cnt_acc = tuple(c_ + jnp.float32(ct) * m_ for c_, m_ in zip(cnt_acc, mcnt))

    pbase = pair * c * h * w

    def osrc(ci):
      ch = c0 + ci
      return ori_h.at[pl.ds(pbase + ch * h * w + r0 * w, nr * w)]

    def ssrc(ci):
      ch = c0 + ci
      return shf_h.at[pl.ds(pbase + ch * h * w + start * w, nr * w)]

    def odst(b):
      return ori_v.at[pl.ds(b * OSTR, nr * w)]

    def sdst(b):
      return shf_v.at[pl.ds(b * SSTR + PAD, nr * w)]

    def issue(ci, b):
      pltpu.async_copy(osrc(ci), odst(b), sem_o[b])
      pltpu.async_copy(ssrc(ci), sdst(b), sem_s[b])

    def wait(b):
      pltpu.make_async_copy(osrc(0), odst(b), sem_o[b]).wait()
      pltpu.make_async_copy(ssrc(0), sdst(b), sem_s[b]).wait()

    def compute(b, acc):
      def row_body(r, racc):
        lyc = jnp.maximum(r + d_off, 0)
        obase = b * OSTR + r * w
        sbase = b * SSTR + PAD + lyc * w - sw
        mbase = r * w
        racc = list(racc)
        for xb in range(nxb):
          o = ori_v[pl.ds(obase + xb * L, L)]
          sv = shf_v[pl.ds(sbase + xb * L, L)]
          m = mv_v[pl.ds(mbase + xb * L, L)]
          d = o - sv
          racc[xb & 3] = racc[xb & 3] + m * d * d
        return tuple(racc)

      return lax.fori_loop(0, nr, row_body, acc)

    if ct == 1:
      pltpu.sync_copy(osrc(0), odst(0))
      pltpu.sync_copy(ssrc(0), sdst(0))
      num_acc = compute(0, num_acc)
    else:
      issue(0, 0)

      def grp_body(g, acc):
        ci0 = 2 * g
        issue(ci0 + 1, 1)
        wait(0)
        acc = compute(0, acc)

        @pl.when(ci0 + 2 < ct)
        def _():
          issue(ci0 + 2, 0)

        wait(1)
        return compute(1, acc)

      num_acc = lax.fori_loop(0, ct // 2, grp_body, num_acc)
    return num_acc, cnt_acc

  zz = (zeros,) * 4
  numf, cntf = do_level(o0, s0, m0, 0, zz, zz)
  numf, cntf = do_level(o1, s1, m1, 1, numf, cntf)
  numf, cntf = do_level(o2, s2, m2, 2, numf, cntf)
  numf, cntf = do_level(o3, s3, mf, 3, numf, cntf)
  numd, cntd = do_level(od, sd, mf, 4, zz, zz)

  def total(t4):
    return (t4[0] + t4[1]) + (t4[2] + t4[3])

  acc_v[pl.ds(0, L)] = total(numf)
  acc_v[pl.ds(L, L)] = total(cntf)
  acc_v[pl.ds(2 * L, L)] = total(numd)
  acc_v[pl.ds(3 * L, L)] = total(cntd)
  pltpu.sync_copy(acc_v, out.at[pl.ds(wid * 4 * L, 4 * L)])


_sc_call = functools.partial(
    pl.kernel,
    out_type=jax.ShapeDtypeStruct((NW * 4 * L,), jnp.float32),
    mesh=plsc.VectorSubcoreMesh(core_axis_name="c", subcore_axis_name="s",
                                num_cores=NC, num_subcores=NS),
    scratch_types=[
        pltpu.VMEM((2 * OSTR,), jnp.float32),
        pltpu.VMEM((2 * SSTR,), jnp.float32),
        pltpu.VMEM((MV_MAX,), jnp.float32),
        pltpu.VMEM((L,), jnp.int32),
        pltpu.VMEM((4 * L,), jnp.float32),
        pltpu.SemaphoreType.DMA,
        pltpu.SemaphoreType.DMA,
        pltpu.SemaphoreType.DMA,
        pltpu.SemaphoreType.DMA,
    ],
)(_sc_body)


def _shift_scalars(shifts):
  shifts = shifts.astype(jnp.int32)
  cols = []
  for (_, h, _, s, _, _) in LEVELS[:4]:
    sh_tab = np.array([int(int(v) * (384.0 / 540.0) / s) for v in range(32, 64)],
                      np.int32)
    sw_tab = np.array([int(int(v) * (512.0 / 960.0) / s) for v in range(32, 64)],
                      np.int32)
    cols.append(jnp.take(jnp.asarray(sh_tab), shifts[:, 0] - 32))
    cols.append(jnp.take(jnp.asarray(sw_tab), shifts[:, 1] - 32))
  cols.append(shifts[:, 0])
  cols.append(shifts[:, 1])
  for _ in range(L - 10):
    cols.append(jnp.zeros((2,), jnp.int32))
  return jnp.stack(cols, axis=1)


@jax.jit
def kernel(depth_preds, shifts, mask, temp_features_0, temp_features_1,
           temp_features_2, temp_features_3):
  feats = [temp_features_0, temp_features_1, temp_features_2, temp_features_3]
  args = []
  for f in feats:
    args.append(f[:2].reshape(-1))
    args.append(f[2:].reshape(-1))
  args.append(depth_preds[:2].reshape(-1))
  args.append(depth_preds[2:].reshape(-1))
  mf = mask[:2, 0].astype(jnp.float32)
  args.append(mf[:, ::8, ::8].reshape(-1))
  args.append(mf[:, ::4, ::4].reshape(-1))
  args.append(mf[:, ::2, ::2].reshape(-1))
  args.append(mf.reshape(-1))
  args.append(_shift_scalars(shifts).reshape(-1))

  parts = _sc_call(*args).reshape(NW, 4, L)
  sums = jnp.sum(parts, axis=(0, 2))
  numf, cntf, numd, cntd = sums[0], sums[1], sums[2], sums[3]
  loss_feat = jnp.where(cntf > 0, numf / jnp.maximum(cntf, 1.0), 0.0)
  loss_pred = jnp.where(cntd > 0, numd / jnp.maximum(cntd, 1.0), 0.0)
  return loss_pred + loss_feat


# speedup vs baseline: 1.4560x; 1.4560x over previous
import functools
import numpy as np
import jax
import jax.numpy as jnp
from jax import lax
from jax.experimental import pallas as pl
from jax.experimental.pallas import tpu as pltpu, tpu_sc as plsc

NC, NS, L = 2, 16, 16
NW = NC * NS

PAD = 64

LEVELS = [
    (256, 48, 64, 8, 48, 16),
    (128, 96, 128, 4, 6, 128),
    (64, 192, 256, 2, 12, 64),
    (32, 384, 512, 1, 24, 32),
    (1, 384, 512, 1, 24, 1),
]

ORI_MAX = max(nr * w for (_, _, w, _, nr, _) in LEVELS)
MV_MAX = ORI_MAX
OSTR = ORI_MAX
SSTR = PAD + ORI_MAX


def _sc_body(f0, f1, f2, f3, fd,
             m0, m1, m2, mf, shsw, out,
             ori_v, shf_v, mv_v, sc_v, acc_v,
             sem_o0, sem_o1, sem_s0, sem_s1):
  wid = lax.axis_index("s") * NC + lax.axis_index("c")
  pair = wid // NS
  t = wid % NS
  lanes = lax.iota(jnp.int32, L)
  zeros = jnp.zeros((L,), jnp.float32)

  pltpu.sync_copy(shsw.at[pl.ds(pair * L, L)], sc_v)
  scv = sc_v[...]

  def get_scal(k):
    return scv[k]

  for b in range(2):
    for i in range(PAD // L):
      shf_v[pl.ds(b * SSTR + i * L, L)] = zeros
  sem_o = (sem_o0, sem_o1)
  sem_s = (sem_s0, sem_s1)

  def do_level(f_h, msk_h, lvl, num_acc, cnt_acc):
    c, h, w, s, nr, ct = LEVELS[lvl]
    nxb = w // L
    sh = get_scal(2 * lvl)
    sw = get_scal(2 * lvl + 1)
    if ct == c:
      c0 = 0
      r0 = t * nr
    else:
      c0 = t * ct
      r0 = 0
    start = jnp.maximum(r0 - sh, 0)
    d_off = r0 - sh - start

    pltpu.sync_copy(msk_h.at[pl.ds((pair * h + r0) * w, nr * w)],
                    ori_v.at[pl.ds(0, nr * w)])

    def mv_row(r, cacc):
      y = r0 + r
      rowf = jnp.where(y >= sh, jnp.float32(1.0), jnp.float32(0.0))
      for xb in range(nxb):
        xs = lanes + (xb * L)
        mval = ori_v[pl.ds(r * w + xb * L, L)]
        colv = jnp.where(xs >= sw, jnp.float32(1.0), jnp.float32(0.0))
        mv = mval * colv * rowf
        mv_v[pl.ds(r * w + xb * L, L)] = mv
        cacc = cacc + mv
      return cacc

    mcnt = lax.fori_loop(0, nr, mv_row, zeros)
    cnt_acc = cnt_acc + jnp.float32(ct) * mcnt

    obase_h = pair * c * h * w
    sbase_h = (2 + pair) * c * h * w

    def osrc(ci):
      ch = c0 + ci
      return f_h.at[pl.ds(obase_h + ch * h * w + r0 * w, nr * w)]

    def ssrc(ci):
      ch = c0 + ci
      return f_h.at[pl.ds(sbase_h + ch * h * w + start * w, nr * w)]

    def odst(b):
      return ori_v.at[pl.ds(b * OSTR, nr * w)]

    def sdst(b):
      return shf_v.at[pl.ds(b * SSTR + PAD, nr * w)]

    def issue(ci, b):
      pltpu.async_copy(osrc(ci), odst(b), sem_o[b])
      pltpu.async_copy(ssrc(ci), sdst(b), sem_s[b])

    def wait(b):
      pltpu.make_async_copy(osrc(0), odst(b), sem_o[b]).wait()
      pltpu.make_async_copy(ssrc(0), sdst(b), sem_s[b]).wait()

    def compute(b, acc):
      def row_body(r, racc):
        lyc = jnp.maximum(r + d_off, 0)
        obase = b * OSTR + r * w
        sbase = b * SSTR + PAD + lyc * w - sw
        mbase = r * w
        for xb in range(nxb):
          o = ori_v[pl.ds(obase + xb * L, L)]
          sv = shf_v[pl.ds(sbase + xb * L, L)]
          m = mv_v[pl.ds(mbase + xb * L, L)]
          d = o - sv
          racc = racc + m * d * d
        return racc

      return lax.fori_loop(0, nr, row_body, acc)

    if ct == 1:
      pltpu.sync_copy(osrc(0), odst(0))
      pltpu.sync_copy(ssrc(0), sdst(0))
      num_acc = compute(0, num_acc)
    else:
      issue(0, 0)

      def grp_body(g, acc):
        ci0 = 2 * g
        issue(ci0 + 1, 1)
        wait(0)
        acc = compute(0, acc)

        @pl.when(ci0 + 2 < ct)
        def _():
          issue(ci0 + 2, 0)

        wait(1)
        return compute(1, acc)

      num_acc = lax.fori_loop(0, ct // 2, grp_body, num_acc)
    return num_acc, cnt_acc

  numf, cntf = do_level(f0, m0, 0, zeros, zeros)
  numf, cntf = do_level(f1, m1, 1, numf, cntf)
  numf, cntf = do_level(f2, m2, 2, numf, cntf)
  numf, cntf = do_level(f3, mf, 3, numf, cntf)
  numd, cntd = do_level(fd, mf, 4, zeros, zeros)

  acc_v[pl.ds(0, L)] = numf
  acc_v[pl.ds(L, L)] = cntf
  acc_v[pl.ds(2 * L, L)] = numd
  acc_v[pl.ds(3 * L, L)] = cntd
  pltpu.sync_copy(acc_v, out.at[pl.ds(wid * 4 * L, 4 * L)])


_sc_call = functools.partial(
    pl.kernel,
    out_type=jax.ShapeDtypeStruct((NW * 4 * L,), jnp.float32),
    mesh=plsc.VectorSubcoreMesh(core_axis_name="c", subcore_axis_name="s",
                                num_cores=NC, num_subcores=NS),
    scratch_types=[
        pltpu.VMEM((2 * OSTR,), jnp.float32),
        pltpu.VMEM((2 * SSTR,), jnp.float32),
        pltpu.VMEM((MV_MAX,), jnp.float32),
        pltpu.VMEM((L,), jnp.int32),
        pltpu.VMEM((4 * L,), jnp.float32),
        pltpu.SemaphoreType.DMA,
        pltpu.SemaphoreType.DMA,
        pltpu.SemaphoreType.DMA,
        pltpu.SemaphoreType.DMA,
    ],
)(_sc_body)


def _shift_scalars(shifts):
  shifts = shifts.astype(jnp.int32)
  cols = []
  for (_, h, _, s, _, _) in LEVELS[:4]:
    sh_tab = np.array([int(int(v) * (384.0 / 540.0) / s) for v in range(32, 64)],
                      np.int32)
    sw_tab = np.array([int(int(v) * (512.0 / 960.0) / s) for v in range(32, 64)],
                      np.int32)
    cols.append(jnp.take(jnp.asarray(sh_tab), shifts[:, 0] - 32))
    cols.append(jnp.take(jnp.asarray(sw_tab), shifts[:, 1] - 32))
  cols.append(shifts[:, 0])
  cols.append(shifts[:, 1])
  for _ in range(L - 10):
    cols.append(jnp.zeros((2,), jnp.int32))
  return jnp.stack(cols, axis=1)


@jax.jit
def kernel(depth_preds, shifts, mask, temp_features_0, temp_features_1,
           temp_features_2, temp_features_3):
  feats = [temp_features_0, temp_features_1, temp_features_2, temp_features_3]
  args = [f.reshape(-1) for f in feats]
  args.append(depth_preds.reshape(-1))
  mf = mask[:2, 0].astype(jnp.float32)
  args.append(mf[:, ::8, ::8].reshape(-1))
  args.append(mf[:, ::4, ::4].reshape(-1))
  args.append(mf[:, ::2, ::2].reshape(-1))
  args.append(mf.reshape(-1))
  args.append(_shift_scalars(shifts).reshape(-1))

  parts = _sc_call(*args).reshape(NW, 4, L)
  sums = jnp.sum(parts, axis=(0, 2))
  numf, cntf, numd, cntd = sums[0], sums[1], sums[2], sums[3]
  loss_feat = jnp.where(cntf > 0, numf / jnp.maximum(cntf, 1.0), 0.0)
  loss_pred = jnp.where(cntd > 0, numd / jnp.maximum(cntd, 1.0), 0.0)
  return loss_pred + loss_feat


# speedup vs baseline: 1.4606x; 1.0032x over previous
import functools
import numpy as np
import jax
import jax.numpy as jnp
from jax import lax
from jax.experimental import pallas as pl
from jax.experimental.pallas import tpu as pltpu, tpu_sc as plsc

NC, NS, L = 2, 16, 16
NW = NC * NS

PAD = 64

LEVELS = [
    (256, 48, 64, 8, 48, 16),
    (128, 96, 128, 4, 6, 128),
    (64, 192, 256, 2, 12, 64),
    (32, 384, 512, 1, 24, 32),
    (1, 384, 512, 1, 24, 1),
]

NRS = 32
W0 = 512
OSTR = max(nr * w for (_, _, w, _, nr, _) in LEVELS)
SSTR = PAD + NRS * W0
MV_MAX = OSTR


def _sc_body(f0, f1, f2, f3, fd,
             m0, m1, m2, mf, shsw, out,
             ori_v, shf_v, mv_v, sc_v, acc_v,
             sem_o0, sem_o1, sem_s0, sem_s1):
  wid = lax.axis_index("s") * NC + lax.axis_index("c")
  pair = wid // NS
  t = wid % NS
  lanes = lax.iota(jnp.int32, L)
  zeros = jnp.zeros((L,), jnp.float32)

  pltpu.sync_copy(shsw.at[pl.ds(pair * L, L)], sc_v)
  scv = sc_v[...]

  def get_scal(k):
    return scv[k]

  for b in range(2):
    for i in range(PAD // L):
      shf_v[pl.ds(b * SSTR + i * L, L)] = zeros
  sem_o = (sem_o0, sem_o1)
  sem_s = (sem_s0, sem_s1)

  def do_level(f_h, msk_h, lvl, num_acc, cnt_acc):
    c, h, w, s, nr, ct = LEVELS[lvl]
    nxb = w // L
    sh = get_scal(2 * lvl)
    sw = get_scal(2 * lvl + 1)
    if ct == c:
      c0 = 0
      r0 = t * nr
    else:
      c0 = t * ct
      r0 = 0
    start = jnp.maximum(r0 - sh, 0)
    d_off = r0 - sh - start

    pltpu.sync_copy(msk_h.at[pl.ds((pair * h + r0) * w, nr * w)],
                    mv_v.at[pl.ds(0, nr * w)])

    def mv_row(r, cacc):
      y = r0 + r
      rowf = jnp.where(y >= sh, jnp.float32(1.0), jnp.float32(0.0))
      for xb in range(nxb):
        xs = lanes + (xb * L)
        mval = mv_v[pl.ds(r * w + xb * L, L)]
        colv = jnp.where(xs >= sw, jnp.float32(1.0), jnp.float32(0.0))
        mv = mval * colv * rowf
        mv_v[pl.ds(r * w + xb * L, L)] = mv
        cacc = cacc + mv
      return cacc

    mcnt = lax.fori_loop(0, nr, mv_row, zeros)
    cnt_acc = cnt_acc + jnp.float32(ct) * mcnt

    obase_h = pair * c * h * w
    sbase_h = (2 + pair) * c * h * w
    doff = d_off

    def osrc(ci):
      ch = c0 + ci
      return f_h.at[pl.ds(obase_h + ch * h * w + r0 * w, nr * w)]

    def ssrc(ci):
      ch = c0 + ci
      return f_h.at[pl.ds(sbase_h + ch * h * w + start * w, nr * w)]

    def odst(b):
      return ori_v.at[pl.ds(b * OSTR, nr * w)]

    def sdst(b):
      return shf_v.at[pl.ds(b * SSTR + PAD, nr * w)]

    def compute(b, acc):
      def row_body(r, racc):
        lyc = jnp.maximum(r + doff, 0)
        obase = b * OSTR + r * w
        sbase = b * SSTR + PAD + lyc * w - sw
        mbase = r * w
        for xb in range(nxb):
          o = ori_v[pl.ds(obase + xb * L, L)]
          sv = shf_v[pl.ds(sbase + xb * L, L)]
          m = mv_v[pl.ds(mbase + xb * L, L)]
          d = o - sv
          racc = racc + m * d * d
        return racc

      return lax.fori_loop(0, nr, row_body, acc)

    def issue(ci, b):
      pltpu.async_copy(osrc(ci), odst(b), sem_o[b])
      pltpu.async_copy(ssrc(ci), sdst(b), sem_s[b])

    def wait(b):
      pltpu.make_async_copy(osrc(0), odst(b), sem_o[b]).wait()
      pltpu.make_async_copy(ssrc(0), sdst(b), sem_s[b]).wait()

    if ct == 1:
      pltpu.sync_copy(osrc(0), odst(0))
      pltpu.sync_copy(ssrc(0), sdst(0))
      num_acc = compute(0, num_acc)
    else:
      issue(0, 0)

      def grp_body(g, acc):
        ci0 = 2 * g
        issue(ci0 + 1, 1)
        wait(0)
        acc = compute(0, acc)

        @pl.when(ci0 + 2 < ct)
        def _():
          issue(ci0 + 2, 0)

        wait(1)
        return compute(1, acc)

      num_acc = lax.fori_loop(0, ct // 2, grp_body, num_acc)
    return num_acc, cnt_acc

  numf, cntf = do_level(f0, m0, 0, zeros, zeros)
  numf, cntf = do_level(f1, m1, 1, numf, cntf)
  numf, cntf = do_level(f2, m2, 2, numf, cntf)
  numf, cntf = do_level(f3, mf, 3, numf, cntf)
  numd, cntd = do_level(fd, mf, 4, zeros, zeros)

  acc_v[pl.ds(0, L)] = numf
  acc_v[pl.ds(L, L)] = cntf
  acc_v[pl.ds(2 * L, L)] = numd
  acc_v[pl.ds(3 * L, L)] = cntd
  pltpu.sync_copy(acc_v, out.at[pl.ds(wid * 4 * L, 4 * L)])


_sc_call = functools.partial(
    pl.kernel,
    out_type=jax.ShapeDtypeStruct((NW * 4 * L,), jnp.float32),
    mesh=plsc.VectorSubcoreMesh(core_axis_name="c", subcore_axis_name="s",
                                num_cores=NC, num_subcores=NS),
    scratch_types=[
        pltpu.VMEM((2 * OSTR,), jnp.float32),
        pltpu.VMEM((2 * SSTR,), jnp.float32),
        pltpu.VMEM((MV_MAX,), jnp.float32),
        pltpu.VMEM((L,), jnp.int32),
        pltpu.VMEM((4 * L,), jnp.float32),
        pltpu.SemaphoreType.DMA,
        pltpu.SemaphoreType.DMA,
        pltpu.SemaphoreType.DMA,
        pltpu.SemaphoreType.DMA,
    ],
)(_sc_body)


def _shift_scalars(shifts):
  shifts = shifts.astype(jnp.int32)
  cols = []
  for (_, h, _, s, _, _) in LEVELS[:4]:
    sh_tab = np.array([int(int(v) * (384.0 / 540.0) / s) for v in range(32, 64)],
                      np.int32)
    sw_tab = np.array([int(int(v) * (512.0 / 960.0) / s) for v in range(32, 64)],
                      np.int32)
    cols.append(jnp.take(jnp.asarray(sh_tab), shifts[:, 0] - 32))
    cols.append(jnp.take(jnp.asarray(sw_tab), shifts[:, 1] - 32))
  cols.append(shifts[:, 0])
  cols.append(shifts[:, 1])
  for _ in range(L - 10):
    cols.append(jnp.zeros((2,), jnp.int32))
  return jnp.stack(cols, axis=1)


@jax.jit
def kernel(depth_preds, shifts, mask, temp_features_0, temp_features_1,
           temp_features_2, temp_features_3):
  args = [temp_features_0.reshape(-1), temp_features_1.reshape(-1),
          temp_features_2.reshape(-1), temp_features_3.reshape(-1),
          depth_preds.reshape(-1)]
  mf = mask[:2, 0].astype(jnp.float32)
  args.append(mf[:, ::8, ::8].reshape(-1))
  args.append(mf[:, ::4, ::4].reshape(-1))
  args.append(mf[:, ::2, ::2].reshape(-1))
  args.append(mf.reshape(-1))
  args.append(_shift_scalars(shifts).reshape(-1))

  parts = _sc_call(*args).reshape(NW, 4, L)
  sums = jnp.sum(parts, axis=(0, 2))
  numf, cntf, numd, cntd = sums[0], sums[1], sums[2], sums[3]
  loss_feat = jnp.where(cntf > 0, numf / jnp.maximum(cntf, 1.0), 0.0)
  loss_pred = jnp.where(cntd > 0, numd / jnp.maximum(cntd, 1.0), 0.0)
  return loss_pred + loss_feat
